# baseline (speedup 1.0000x reference)
import functools

import jax
import jax.numpy as jnp
from jax import lax
from jax.experimental import pallas as pl
from jax.experimental.pallas import tpu as pltpu
from jax.experimental.pallas import tpu_sc as plsc

S = 2048
B = 4
D = 1024
S_TC = 1024
S_SC = S - S_TC
NC = 2
NS = 16
NW = NC * NS
ROWS_PER_W = S_SC // NW
CHUNK = 8
NBLK = ROWS_PER_W // CHUNK
NBUF = 3
K = 1
LANES = 16
JPR = D // LANES
TC_BS = 8


def _sc_pos_add(x, table):
    mesh = plsc.VectorSubcoreMesh(core_axis_name="c", subcore_axis_name="s")

    @functools.partial(
        pl.kernel,
        mesh=mesh,
        out_type=jax.ShapeDtypeStruct((S_SC, B, D), jnp.float32),
        scratch_types=[
            pltpu.VMEM((NBUF, CHUNK, B, D), jnp.float32),
            pltpu.VMEM((NBUF, CHUNK, D), jnp.float32),
            [pltpu.SemaphoreType.DMA] * NBUF,
            [pltpu.SemaphoreType.DMA] * NBUF,
        ],
    )
    def k(x_hbm, t_hbm, out_hbm, xbuf, tbuf, lsem, ssem):
        wid = lax.axis_index("s") * NC + lax.axis_index("c")
        base = S_TC + wid * ROWS_PER_W
        obase = wid * ROWS_PER_W

        def start_load(blk):
            slot = blk % NBUF
            r0 = base + blk * CHUNK
            pltpu.async_copy(
                x_hbm.at[pl.ds(r0, CHUNK)], xbuf.at[slot], lsem[slot])
            pltpu.async_copy(
                t_hbm.at[pl.ds(r0, CHUNK)], tbuf.at[slot], lsem[slot])

        def wait_load(slot):
            pltpu.make_async_copy(
                x_hbm.at[pl.ds(0, CHUNK)], xbuf.at[slot], lsem[slot]).wait()
            pltpu.make_async_copy(
                t_hbm.at[pl.ds(0, CHUNK)], tbuf.at[slot], lsem[slot]).wait()

        def start_store(blk):
            slot = blk % NBUF
            pltpu.async_copy(
                xbuf.at[slot], out_hbm.at[pl.ds(obase + blk * CHUNK, CHUNK)],
                ssem[slot])

        def wait_store(slot):
            pltpu.make_async_copy(
                xbuf.at[slot], out_hbm.at[pl.ds(0, CHUNK)], ssem[slot]).wait()

        def compute(slot):
            xb = xbuf.at[slot]
            tb = tbuf.at[slot]

            @pl.loop(0, CHUNK)
            def _(i):
                @plsc.parallel_loop(0, JPR, unroll=4)
                def _(j):
                    jo = j * LANES
                    t = tb[i, pl.ds(jo, LANES)]
                    for b in range(B):
                        xb[i, b, pl.ds(jo, LANES)] += t

        for blk in range(NBUF):
            start_load(blk)
        for blk in range(NBLK):
            slot = blk % NBUF
            wait_load(slot)
            compute(slot)
            start_store(blk)
            reload = blk - K + NBUF
            if blk >= K and reload < NBLK:
                wait_store(reload % NBUF)
                start_load(reload)
        for blk in range(max(NBLK - NBUF, 0), NBLK):
            wait_store(blk % NBUF)

    return k(x, table)


def _tc_body(x_ref, t_ref, o_ref):
    o_ref[...] = x_ref[...] + t_ref[...][:, None, :]


def _tc_pos_add(x, table):
    grid = (S_TC // TC_BS,)
    return pl.pallas_call(
        _tc_body,
        grid=grid,
        in_specs=[
            pl.BlockSpec((TC_BS, B, D), lambda i: (i, 0, 0)),
            pl.BlockSpec((TC_BS, D), lambda i: (i, 0)),
        ],
        out_specs=pl.BlockSpec((TC_BS, B, D), lambda i: (i, 0, 0)),
        out_shape=jax.ShapeDtypeStruct((S_TC, B, D), jnp.float32),
    )(x, table)


def kernel(x, pos_emb_table):
    sc_out = _sc_pos_add(x, pos_emb_table)
    tc_out = _tc_pos_add(x, pos_emb_table)
    return jnp.concatenate([tc_out, sc_out], axis=0)


# speedup vs baseline: 2.3433x; 2.3433x over previous
import functools

import jax
import jax.numpy as jnp
from jax import lax
from jax.experimental import pallas as pl
from jax.experimental.pallas import tpu as pltpu
from jax.experimental.pallas import tpu_sc as plsc

S = 2048
B = 4
D = 1024
NC = 2
NS = 16
NW = NC * NS
ROWS_PER_W = S // NW
CHUNK = 8
NBLK = ROWS_PER_W // CHUNK
NBUF = 3
K = 2
LANES = 16
JPR = D // LANES


def _sc_pos_add(x, table):
    mesh = plsc.VectorSubcoreMesh(core_axis_name="c", subcore_axis_name="s")

    @functools.partial(
        pl.kernel,
        mesh=mesh,
        out_type=jax.ShapeDtypeStruct((S, B, D), jnp.float32),
        scratch_types=[
            pltpu.VMEM((NBUF, CHUNK, B, D), jnp.float32),
            pltpu.VMEM((NBUF, CHUNK, D), jnp.float32),
            [pltpu.SemaphoreType.DMA] * NBUF,
            [pltpu.SemaphoreType.DMA] * NBUF,
        ],
    )
    def k(x_hbm, t_hbm, out_hbm, xbuf, tbuf, lsem, ssem):
        wid = lax.axis_index("s") * NC + lax.axis_index("c")
        base = wid * ROWS_PER_W

        def start_load(blk):
            slot = blk % NBUF
            r0 = base + blk * CHUNK
            pltpu.async_copy(
                x_hbm.at[pl.ds(r0, CHUNK)], xbuf.at[slot], lsem[slot])
            pltpu.async_copy(
                t_hbm.at[pl.ds(r0, CHUNK)], tbuf.at[slot], lsem[slot])

        def wait_load(slot):
            pltpu.make_async_copy(
                x_hbm.at[pl.ds(0, CHUNK)], xbuf.at[slot], lsem[slot]).wait()
            pltpu.make_async_copy(
                t_hbm.at[pl.ds(0, CHUNK)], tbuf.at[slot], lsem[slot]).wait()

        def start_store(blk):
            slot = blk % NBUF
            pltpu.async_copy(
                xbuf.at[slot], out_hbm.at[pl.ds(base + blk * CHUNK, CHUNK)],
                ssem[slot])

        def wait_store(slot):
            pltpu.make_async_copy(
                xbuf.at[slot], out_hbm.at[pl.ds(0, CHUNK)], ssem[slot]).wait()

        def compute(slot):
            xb = xbuf.at[slot]
            tb = tbuf.at[slot]

            @pl.loop(0, CHUNK)
            def _(i):
                @plsc.parallel_loop(0, JPR, unroll=4)
                def _(j):
                    jo = j * LANES
                    t = tb[i, pl.ds(jo, LANES)]
                    for b in range(B):
                        xb[i, b, pl.ds(jo, LANES)] += t

        for blk in range(NBUF):
            start_load(blk)
        for blk in range(NBLK):
            slot = blk % NBUF
            wait_load(slot)
            compute(slot)
            start_store(blk)
            reload = blk - K + NBUF
            if blk >= K and reload < NBLK:
                wait_store(reload % NBUF)
                start_load(reload)
        for blk in range(max(NBLK - NBUF, 0), NBLK):
            wait_store(blk % NBUF)

    return k(x, table)


def kernel(x, pos_emb_table):
    return _sc_pos_add(x, pos_emb_table)


# speedup vs baseline: 2.7184x; 1.1600x over previous
import functools

import jax
import jax.numpy as jnp
from jax import lax
from jax.experimental import pallas as pl
from jax.experimental.pallas import tpu as pltpu
from jax.experimental.pallas import tpu_sc as plsc

S = 2048
B = 4
D = 1024
NC = 2
NS = 16
NW = NC * NS
ROWS_PER_W = S // NW
CHUNK = 8
NBLK = ROWS_PER_W // CHUNK
NBUF = 3
K = 1
LANES = 16
JPR = D // LANES


def _sc_pos_add(x, table):
    mesh = plsc.VectorSubcoreMesh(core_axis_name="c", subcore_axis_name="s")

    @functools.partial(
        pl.kernel,
        mesh=mesh,
        out_type=jax.ShapeDtypeStruct((S, B, D), jnp.float32),
        scratch_types=[
            pltpu.VMEM((NBUF, CHUNK, B, D), jnp.float32),
            pltpu.VMEM((NBUF, CHUNK, D), jnp.float32),
            [pltpu.SemaphoreType.DMA] * NBUF,
            [pltpu.SemaphoreType.DMA] * NBUF,
        ],
    )
    def k(x_hbm, t_hbm, out_hbm, xbuf, tbuf, lsem, ssem):
        wid = lax.axis_index("s") * NC + lax.axis_index("c")
        base = wid * ROWS_PER_W

        def start_load(blk):
            slot = blk % NBUF
            r0 = base + blk * CHUNK
            pltpu.async_copy(
                x_hbm.at[pl.ds(r0, CHUNK)], xbuf.at[slot], lsem[slot])
            pltpu.async_copy(
                t_hbm.at[pl.ds(r0, CHUNK)], tbuf.at[slot], lsem[slot])

        def wait_load(slot):
            pltpu.make_async_copy(
                x_hbm.at[pl.ds(0, CHUNK)], xbuf.at[slot], lsem[slot]).wait()
            pltpu.make_async_copy(
                t_hbm.at[pl.ds(0, CHUNK)], tbuf.at[slot], lsem[slot]).wait()

        H = CHUNK // 2

        def start_store_half(blk, h):
            slot = blk % NBUF
            pltpu.async_copy(
                xbuf.at[slot, pl.ds(h * H, H)],
                out_hbm.at[pl.ds(base + blk * CHUNK + h * H, H)],
                ssem[slot])

        def wait_store(slot):
            for h in range(2):
                pltpu.make_async_copy(
                    xbuf.at[slot, pl.ds(0, H)], out_hbm.at[pl.ds(0, H)],
                    ssem[slot]).wait()

        def compute_half(slot, h):
            xb = xbuf.at[slot]
            tb = tbuf.at[slot]

            @pl.loop(h * H, (h + 1) * H)
            def _(i):
                @plsc.parallel_loop(0, JPR, unroll=4)
                def _(j):
                    jo = j * LANES
                    t = tb[i, pl.ds(jo, LANES)]
                    for b in range(B):
                        xb[i, b, pl.ds(jo, LANES)] += t

        for blk in range(NBUF):
            start_load(blk)
        for blk in range(NBLK):
            slot = blk % NBUF
            wait_load(slot)
            compute_half(slot, 0)
            start_store_half(blk, 0)
            compute_half(slot, 1)
            start_store_half(blk, 1)
            reload = blk - K + NBUF
            if blk >= K and reload < NBLK:
                wait_store(reload % NBUF)
                start_load(reload)
        for blk in range(max(NBLK - NBUF, 0), NBLK):
            wait_store(blk % NBUF)

    return k(x, table)


def kernel(x, pos_emb_table):
    return _sc_pos_add(x, pos_emb_table)


# speedup vs baseline: 2.7770x; 1.0216x over previous
import functools

import jax
import jax.numpy as jnp
from jax import lax
from jax.experimental import pallas as pl
from jax.experimental.pallas import tpu as pltpu
from jax.experimental.pallas import tpu_sc as plsc

S = 2048
B = 4
D = 1024
NC = 2
NS = 16
NW = NC * NS
ROWS_PER_W = S // NW
CHUNK = 4
NBLK = ROWS_PER_W // CHUNK
NBUF = 6
K = 2
LANES = 16
JPR = D // LANES


def _sc_pos_add(x, table):
    mesh = plsc.VectorSubcoreMesh(core_axis_name="c", subcore_axis_name="s")

    @functools.partial(
        pl.kernel,
        mesh=mesh,
        out_type=jax.ShapeDtypeStruct((S, B, D), jnp.float32),
        scratch_types=[
            pltpu.VMEM((NBUF, CHUNK, B, D), jnp.float32),
            pltpu.VMEM((NBUF, CHUNK, D), jnp.float32),
            [pltpu.SemaphoreType.DMA] * NBUF,
            [pltpu.SemaphoreType.DMA] * NBUF,
        ],
    )
    def k(x_hbm, t_hbm, out_hbm, xbuf, tbuf, lsem, ssem):
        wid = lax.axis_index("s") * NC + lax.axis_index("c")
        base = wid * ROWS_PER_W

        def start_load(blk):
            slot = blk % NBUF
            r0 = base + blk * CHUNK
            pltpu.async_copy(
                x_hbm.at[pl.ds(r0, CHUNK)], xbuf.at[slot], lsem[slot])
            pltpu.async_copy(
                t_hbm.at[pl.ds(r0, CHUNK)], tbuf.at[slot], lsem[slot])

        def wait_load(slot):
            pltpu.make_async_copy(
                x_hbm.at[pl.ds(0, CHUNK)], xbuf.at[slot], lsem[slot]).wait()
            pltpu.make_async_copy(
                t_hbm.at[pl.ds(0, CHUNK)], tbuf.at[slot], lsem[slot]).wait()

        def start_store(blk):
            slot = blk % NBUF
            pltpu.async_copy(
                xbuf.at[slot], out_hbm.at[pl.ds(base + blk * CHUNK, CHUNK)],
                ssem[slot])

        def wait_store(slot):
            pltpu.make_async_copy(
                xbuf.at[slot], out_hbm.at[pl.ds(0, CHUNK)], ssem[slot]).wait()

        def compute(slot):
            xb = xbuf.at[slot]
            tb = tbuf.at[slot]

            @pl.loop(0, CHUNK)
            def _(i):
                @plsc.parallel_loop(0, JPR, unroll=4)
                def _(j):
                    jo = j * LANES
                    t = tb[i, pl.ds(jo, LANES)]
                    for b in range(B):
                        xb[i, b, pl.ds(jo, LANES)] += t

        for blk in range(NBUF):
            start_load(blk)
        for blk in range(NBLK):
            slot = blk % NBUF
            wait_load(slot)
            compute(slot)
            start_store(blk)
            reload = blk - K + NBUF
            if blk >= K and reload < NBLK:
                wait_store(reload % NBUF)
                start_load(reload)
        for blk in range(max(NBLK - NBUF, 0), NBLK):
            wait_store(blk % NBUF)

    return k(x, table)


def kernel(x, pos_emb_table):
    return _sc_pos_add(x, pos_emb_table)


# speedup vs baseline: 2.8088x; 1.0115x over previous
import functools

import jax
import jax.numpy as jnp
from jax import lax
from jax.experimental import pallas as pl
from jax.experimental.pallas import tpu as pltpu
from jax.experimental.pallas import tpu_sc as plsc

S = 2048
B = 4
D = 1024
NC = 2
NS = 16
NW = NC * NS
ROWS_PER_W = S // NW
CHUNK = 4
NBLK = ROWS_PER_W // CHUNK
NBUF = 6
K = 1
LANES = 16
JPR = D // LANES


def _sc_pos_add(x, table):
    mesh = plsc.VectorSubcoreMesh(core_axis_name="c", subcore_axis_name="s")

    @functools.partial(
        pl.kernel,
        mesh=mesh,
        out_type=jax.ShapeDtypeStruct((S, B, D), jnp.float32),
        scratch_types=[
            pltpu.VMEM((NBUF, CHUNK, B, D), jnp.float32),
            pltpu.VMEM((NBUF, CHUNK, D), jnp.float32),
            [pltpu.SemaphoreType.DMA] * NBUF,
            [pltpu.SemaphoreType.DMA] * NBUF,
        ],
    )
    def k(x_hbm, t_hbm, out_hbm, xbuf, tbuf, lsem, ssem):
        wid = lax.axis_index("s") * NC + lax.axis_index("c")
        base = wid * ROWS_PER_W

        def start_load(blk):
            slot = blk % NBUF
            r0 = base + blk * CHUNK
            pltpu.async_copy(
                x_hbm.at[pl.ds(r0, CHUNK)], xbuf.at[slot], lsem[slot])
            pltpu.async_copy(
                t_hbm.at[pl.ds(r0, CHUNK)], tbuf.at[slot], lsem[slot])

        def wait_load(slot):
            pltpu.make_async_copy(
                x_hbm.at[pl.ds(0, CHUNK)], xbuf.at[slot], lsem[slot]).wait()
            pltpu.make_async_copy(
                t_hbm.at[pl.ds(0, CHUNK)], tbuf.at[slot], lsem[slot]).wait()

        def start_store(blk):
            slot = blk % NBUF
            pltpu.async_copy(
                xbuf.at[slot], out_hbm.at[pl.ds(base + blk * CHUNK, CHUNK)],
                ssem[slot])

        def wait_store(slot):
            pltpu.make_async_copy(
                xbuf.at[slot], out_hbm.at[pl.ds(0, CHUNK)], ssem[slot]).wait()

        def compute(slot):
            xb = xbuf.at[slot]
            tb = tbuf.at[slot]

            @pl.loop(0, CHUNK)
            def _(i):
                @plsc.parallel_loop(0, JPR, unroll=4)
                def _(j):
                    jo = j * LANES
                    t = tb[i, pl.ds(jo, LANES)]
                    for b in range(B):
                        xb[i, b, pl.ds(jo, LANES)] += t

        for blk in range(NBUF):
            start_load(blk)
        for blk in range(NBLK):
            slot = blk % NBUF
            wait_load(slot)
            compute(slot)
            start_store(blk)
            reload = blk - K + NBUF
            if blk >= K and reload < NBLK:
                wait_store(reload % NBUF)
                start_load(reload)
        for blk in range(max(NBLK - NBUF, 0), NBLK):
            wait_store(blk % NBUF)

    return k(x, table)


def kernel(x, pos_emb_table):
    return _sc_pos_add(x, pos_emb_table)
